# Initial kernel scaffold; baseline (speedup 1.0000x reference)
import jax
import jax.numpy as jnp
from jax.experimental import pallas as pl


def kernel(node_pos, full_edge_attr, full_edge_index, edge_index, batch, n):
    raise NotImplementedError("write your pallas kernel here")


# speedup vs baseline: 105.9884x; 105.9884x over previous
import functools

import jax
import jax.numpy as jnp
from jax import lax
from jax.experimental import pallas as pl
from jax.experimental.pallas import tpu as pltpu
from jax.experimental.pallas import tpu_sc as plsc

_NC = 2
_NS = 16
_L = 16

_N_PAD = 102400
_NPT = _N_PAD // _NS

_LN2 = 0.6931471805599453
_SQRT2 = 1.4142135623730951


def _vlog(x):
    xi = plsc.bitcast(x, jnp.int32)
    e = (lax.shift_right_logical(xi, 23) & 0xFF) - 127
    mi = (xi & 0x007FFFFF) | 0x3F800000
    m = plsc.bitcast(mi, jnp.float32)
    big = m > _SQRT2
    m = jnp.where(big, m * 0.5, m)
    e = e + jnp.where(big, 1, 0)
    t = m - 1.0
    z = t / (2.0 + t)
    z2 = z * z
    lnm = 2.0 * z * (1.0 + z2 * (1.0 / 3.0 + z2 * (1.0 / 5.0 + z2 * (1.0 / 7.0))))
    return e.astype(jnp.float32) * _LN2 + lnm


def _sc_body(posx, posy, batf, attrf, src2, dst2, cbv,
             outA, outB, outQ,
             zb, ab, sb, db, pb, ssb, sdb,
             gxs, gys, gvs, ggf, gxd, gyd, gvd,
             accA, accB, accQ, cbb, sem,
             ssum_s, ssum_d, pxT, pyT, gfT):
    c = lax.axis_index("c")
    s = lax.axis_index("s")
    w = c * _NS + s

    rows = src2.shape[0]
    nchunks = rows // 8

    lanes = lax.iota(jnp.int32, _L)
    z16f = jnp.zeros((_L,), jnp.float32)

    @pl.loop(0, _NPT // _L)
    def _(i):
        zb[pl.ds(i * _L, _L)] = z16f

    nb = s * _NPT
    pltpu.sync_copy(zb, ssum_s.at[pl.ds(nb, _NPT)])
    pltpu.sync_copy(zb, ssum_d.at[pl.ds(nb, _NPT)])
    pltpu.sync_copy(posx.at[pl.ds(nb, _NPT)], pxT.at[pl.ds(nb, _NPT)])
    pltpu.sync_copy(posy.at[pl.ds(nb, _NPT)], pyT.at[pl.ds(nb, _NPT)])
    pltpu.sync_copy(batf.at[pl.ds(nb, _NPT)], gfT.at[pl.ds(nb, _NPT)])

    @pl.loop(0, 256 // _L)
    def _(i):
        accA[pl.ds(i * _L, _L)] = z16f
        accB[pl.ds(i * _L, _L)] = z16f
        accQ[pl.ds(i * _L, _L)] = z16f

    pltpu.sync_copy(cbv, cbb)
    plsc.subcore_barrier()

    base1 = nchunks // _NS
    rem1 = nchunks % _NS
    st1 = s * base1 + jnp.minimum(s, rem1)
    cnt1 = base1 + jnp.where(s < rem1, 1, 0)

    @pl.loop(st1, st1 + cnt1)
    def _(ch):
        r0 = ch * 8
        pltpu.sync_copy(attrf.at[pl.ds(r0 * 128, 1024)], ab)
        pltpu.sync_copy(src2.at[pl.ds(r0, 8)], sb)
        pltpu.sync_copy(dst2.at[pl.ds(r0, 8)], db)

        @pl.loop(0, 1024 // _L)
        def _(v):
            att = ab[pl.ds(v * _L, _L)]
            pb[pl.ds(v * _L, _L)] = jnp.exp(att * -0.5)

        @pl.loop(0, 8)
        def _(j):
            pltpu.sync_copy(pb.at[pl.ds(j * 128, 128)],
                            ssum_s.at[sb.at[j]], add=True)
            pltpu.sync_copy(pb.at[pl.ds(j * 128, 128)],
                            ssum_d.at[db.at[j]], add=True)

    plsc.subcore_barrier()

    pltpu.sync_copy(ssum_s.at[pl.ds(nb, _NPT)], ssb)
    pltpu.sync_copy(ssum_d.at[pl.ds(nb, _NPT)], sdb)

    @pl.loop(0, _NPT // _L)
    def _(v):
        sl = pl.ds(v * _L, _L)
        ssb[sl] = 1.0 / ssb[sl]
        sdb[sl] = 1.0 / sdb[sl]

    pltpu.sync_copy(ssb, ssum_s.at[pl.ds(nb, _NPT)])
    pltpu.sync_copy(sdb, ssum_d.at[pl.ds(nb, _NPT)])
    plsc.subcore_barrier()

    nw = _NC * _NS
    base2 = nchunks // nw
    rem2 = nchunks % nw
    st2 = w * base2 + jnp.minimum(w, rem2)
    cnt2 = base2 + jnp.where(w < rem2, 1, 0)

    @pl.loop(st2, st2 + cnt2)
    def _(ch):
        r0 = ch * 8
        pltpu.sync_copy(attrf.at[pl.ds(r0 * 128, 1024)], ab)
        pltpu.sync_copy(src2.at[pl.ds(r0, 8)], sb)
        pltpu.sync_copy(dst2.at[pl.ds(r0, 8)], db)

        @pl.loop(0, 8)
        def _(j):
            si = sb.at[j]
            di = db.at[j]
            cps = (
                pltpu.async_copy(pxT.at[si], gxs, sem),
                pltpu.async_copy(pyT.at[si], gys, sem),
                pltpu.async_copy(ssum_s.at[si], gvs, sem),
                pltpu.async_copy(gfT.at[si], ggf, sem),
                pltpu.async_copy(pxT.at[di], gxd, sem),
                pltpu.async_copy(pyT.at[di], gyd, sem),
                pltpu.async_copy(ssum_d.at[di], gvd, sem),
            )
            for cp in cps:
                cp.wait()

            @pl.loop(0, 128 // _L)
            def _(v):
                sl = pl.ds(v * _L, _L)
                att = ab[pl.ds(j * 128 + v * _L, _L)]
                g = ggf[sl].astype(jnp.int32)
                cb = plsc.load_gather(cbb, [g])
                lp = att * -0.5
                p = jnp.exp(lp)
                t = (gvs[sl] + gvd[sl]) * cb
                pn = p * t
                dx = gxs[sl] - gxd[sl]
                dy = gys[sl] - gyd[sl]
                w1 = 1.0 + dx * dx + dy * dy
                qun = 1.0 / w1
                ekl = pn * (lp + _vlog(t) + _vlog(w1))
                fl = g * _L + lanes
                plsc.addupdate_scatter(accA, [fl], ekl)
                plsc.addupdate_scatter(accB, [fl], pn)
                plsc.addupdate_scatter(accQ, [fl], qun)

    pltpu.sync_copy(accA, outA.at[c, s])
    pltpu.sync_copy(accB, outB.at[c, s])
    pltpu.sync_copy(accQ, outQ.at[c, s])


def _fin_body(pA_ref, pB_ref, pQ_ref, o_ref):
    RA = pA_ref[...]
    RB = pB_ref[...]
    RQ = pQ_ref[...]
    col = lax.broadcasted_iota(jnp.int32, RA.shape, 1)
    gcol = col // _L
    ngraphs = 256 // _L
    total = jnp.float32(0.0)
    for g in range(ngraphs):
        m = gcol == g
        Ag = jnp.sum(jnp.where(m, RA, 0.0))
        Bg = jnp.sum(jnp.where(m, RB, 0.0))
        Qg = jnp.sum(jnp.where(m, RQ, 0.0))
        gk = jnp.where(Bg > 0.0, Ag + Bg * jnp.log(Qg), 0.0)
        total = total + gk
    o_ref[0, 0] = total / jnp.float32(ngraphs)


@jax.jit
def kernel(node_pos, full_edge_attr, full_edge_index, edge_index, batch, n):
    del edge_index
    n_nodes = node_pos.shape[0]
    n_edges = full_edge_index.shape[1]
    rows = n_edges // 128

    attrf = full_edge_attr[:, 0]
    src2 = full_edge_index[0].reshape(rows, 128)
    dst2 = full_edge_index[1].reshape(rows, 128)
    pad = _N_PAD - n_nodes
    posx = jnp.pad(node_pos[:, 0], (0, pad))
    posy = jnp.pad(node_pos[:, 1], (0, pad))
    batf = jnp.pad(batch.astype(jnp.float32), (0, pad))
    cbv = 1.0 / (2.0 * n.astype(jnp.float32))

    mesh = plsc.VectorSubcoreMesh(
        core_axis_name="c", subcore_axis_name="s",
        num_cores=_NC, num_subcores=_NS)

    sc = pl.kernel(
        _sc_body,
        out_type=[jax.ShapeDtypeStruct((_NC, _NS, 256), jnp.float32)] * 3,
        mesh=mesh,
        compiler_params=pltpu.CompilerParams(needs_layout_passes=False),
        scratch_types=[
            pltpu.VMEM((_NPT,), jnp.float32),
            pltpu.VMEM((1024,), jnp.float32),
            pltpu.VMEM((8, 128), jnp.int32),
            pltpu.VMEM((8, 128), jnp.int32),
            pltpu.VMEM((1024,), jnp.float32),
            pltpu.VMEM((_NPT,), jnp.float32),
            pltpu.VMEM((_NPT,), jnp.float32),
            pltpu.VMEM((128,), jnp.float32),
            pltpu.VMEM((128,), jnp.float32),
            pltpu.VMEM((128,), jnp.float32),
            pltpu.VMEM((128,), jnp.float32),
            pltpu.VMEM((128,), jnp.float32),
            pltpu.VMEM((128,), jnp.float32),
            pltpu.VMEM((128,), jnp.float32),
            pltpu.VMEM((256,), jnp.float32),
            pltpu.VMEM((256,), jnp.float32),
            pltpu.VMEM((256,), jnp.float32),
            pltpu.VMEM((16,), jnp.float32),
            pltpu.SemaphoreType.DMA,
            pltpu.VMEM_SHARED((_N_PAD,), jnp.float32),
            pltpu.VMEM_SHARED((_N_PAD,), jnp.float32),
            pltpu.VMEM_SHARED((_N_PAD,), jnp.float32),
            pltpu.VMEM_SHARED((_N_PAD,), jnp.float32),
            pltpu.VMEM_SHARED((_N_PAD,), jnp.float32),
        ],
    )
    pA, pB, pQ = sc(posx, posy, batf, attrf, src2, dst2, cbv)

    fin = pl.pallas_call(
        _fin_body,
        out_shape=jax.ShapeDtypeStruct((1, 1), jnp.float32),
        out_specs=pl.BlockSpec(memory_space=pltpu.SMEM),
    )
    res = fin(pA.reshape(_NC * _NS, 256),
              pB.reshape(_NC * _NS, 256),
              pQ.reshape(_NC * _NS, 256))
    return res[0, 0]


# speedup vs baseline: 127.5120x; 1.2031x over previous
import functools

import jax
import jax.numpy as jnp
from jax import lax
from jax.experimental import pallas as pl
from jax.experimental.pallas import tpu as pltpu
from jax.experimental.pallas import tpu_sc as plsc

_NC = 2
_NS = 16
_L = 16

_N_PAD = 102400
_NPT = _N_PAD // _NS

_LN2 = 0.6931471805599453
_SQRT2 = 1.4142135623730951


def _vlog(x):
    xi = plsc.bitcast(x, jnp.int32)
    e = (lax.shift_right_logical(xi, 23) & 0xFF) - 127
    mi = (xi & 0x007FFFFF) | 0x3F800000
    m = plsc.bitcast(mi, jnp.float32)
    big = m > _SQRT2
    m = jnp.where(big, m * 0.5, m)
    e = e + jnp.where(big, 1, 0)
    t = m - 1.0
    z = t / (2.0 + t)
    z2 = z * z
    lnm = 2.0 * z * (1.0 + z2 * (1.0 / 3.0 + z2 * (1.0 / 5.0 + z2 * (1.0 / 7.0))))
    return e.astype(jnp.float32) * _LN2 + lnm


def _sc_body(posx, posy, attrf, srcf32, src2, dst2, cbv, cumv,
             outA, outB, outQ,
             zb, ab, sb, db, pb, sbf,
             ssb, sdb,
             g0, g1, accA, accB, accQ, cbb, cumb,
             semg0, semg1,
             sem_s0, sem_s1, sem_s2, sem_s3,
             sem_d0, sem_d1, sem_d2, sem_d3,
             ss0, ss1, ss2, ss3, sd0, sd1, sd2, sd3, pxT, pyT):
    sstabs = (ss0, ss1, ss2, ss3)
    sdtabs = (sd0, sd1, sd2, sd3)
    ssems = (sem_s0, sem_s1, sem_s2, sem_s3)
    dsems = (sem_d0, sem_d1, sem_d2, sem_d3)
    c = lax.axis_index("c")
    s = lax.axis_index("s")
    w = c * _NS + s

    rows = src2.shape[0]
    nchunks = rows // 8

    lanes = lax.iota(jnp.int32, _L)
    z16f = jnp.zeros((_L,), jnp.float32)

    @pl.loop(0, _NPT // _L)
    def _(i):
        zb[pl.ds(i * _L, _L)] = z16f

    nb = s * _NPT
    for tab in sstabs + sdtabs:
        pltpu.sync_copy(zb, tab.at[pl.ds(nb, _NPT)])
    pltpu.sync_copy(posx.at[pl.ds(nb, _NPT)], pxT.at[pl.ds(nb, _NPT)])
    pltpu.sync_copy(posy.at[pl.ds(nb, _NPT)], pyT.at[pl.ds(nb, _NPT)])

    @pl.loop(0, 256 // _L)
    def _(i):
        accA[pl.ds(i * _L, _L)] = z16f
        accB[pl.ds(i * _L, _L)] = z16f
        accQ[pl.ds(i * _L, _L)] = z16f

    pltpu.sync_copy(cbv, cbb)
    pltpu.sync_copy(cumv, cumb)
    plsc.subcore_barrier()

    base1 = nchunks // _NS
    rem1 = nchunks % _NS
    st1 = s * base1 + jnp.minimum(s, rem1)
    cnt1 = base1 + jnp.where(s < rem1, 1, 0)

    @pl.loop(st1, st1 + cnt1)
    def _(ch):
        r0 = ch * 8
        pltpu.sync_copy(attrf.at[pl.ds(r0 * 128, 1024)], ab)
        pltpu.sync_copy(src2.at[pl.ds(r0, 8)], sb)
        pltpu.sync_copy(dst2.at[pl.ds(r0, 8)], db)

        @pl.loop(0, 1024 // _L)
        def _(v):
            att = ab[pl.ds(v * _L, _L)]
            pb[pl.ds(v * _L, _L)] = jnp.exp(att * -0.5)

        sdesc = [None] * 4
        ddesc = [None] * 4
        for j in range(8):
            k = j % 4
            if sdesc[k] is not None:
                sdesc[k].wait()
                ddesc[k].wait()
            pj = pb.at[pl.ds(j * 128, 128)]
            sdesc[k] = pltpu.async_copy(pj, sstabs[k].at[sb.at[j]],
                                        ssems[k], add=True)
            ddesc[k] = pltpu.async_copy(pj, sdtabs[k].at[db.at[j]],
                                        dsems[k], add=True)
        for k in range(4):
            sdesc[k].wait()
            ddesc[k].wait()

    plsc.subcore_barrier()

    nsl = pl.ds(nb, _NPT)
    pltpu.sync_copy(ss0.at[nsl], ssb)
    pltpu.sync_copy(sd0.at[nsl], sdb)
    for tab, acc in ((ss1, ssb), (ss2, ssb), (ss3, ssb),
                     (sd1, sdb), (sd2, sdb), (sd3, sdb)):
        pltpu.sync_copy(tab.at[nsl], zb)

        @pl.loop(0, _NPT // _L)
        def _(v):
            sl = pl.ds(v * _L, _L)
            acc[sl] = acc[sl] + zb[sl]

    @pl.loop(0, _NPT // _L)
    def _(v):
        sl = pl.ds(v * _L, _L)
        ssb[sl] = 1.0 / ssb[sl]
        sdb[sl] = 1.0 / sdb[sl]

    pltpu.sync_copy(ssb, ss0.at[nsl])
    pltpu.sync_copy(sdb, sd0.at[nsl])
    plsc.subcore_barrier()

    cums = [plsc.load_gather(cumb, [jnp.full((_L,), k, jnp.int32)])
            for k in range(1, 16)]

    nw = _NC * _NS
    base2 = nchunks // nw
    rem2 = nchunks % nw
    st2 = w * base2 + jnp.minimum(w, rem2)
    cnt2 = base2 + jnp.where(w < rem2, 1, 0)

    gsets = (g0, g1)
    gsems = (semg0, semg1)

    def fire(j, gset, gsem):
        si = sb.at[j]
        di = db.at[j]
        return (
            pltpu.async_copy(pxT.at[si], gset.at[pl.ds(0, 128)], gsem),
            pltpu.async_copy(pyT.at[si], gset.at[pl.ds(128, 128)], gsem),
            pltpu.async_copy(ss0.at[si], gset.at[pl.ds(256, 128)], gsem),
            pltpu.async_copy(pxT.at[di], gset.at[pl.ds(384, 128)], gsem),
            pltpu.async_copy(pyT.at[di], gset.at[pl.ds(512, 128)], gsem),
            pltpu.async_copy(sd0.at[di], gset.at[pl.ds(640, 128)], gsem),
        )

    @pl.loop(st2, st2 + cnt2)
    def _(ch):
        r0 = ch * 8
        pltpu.sync_copy(attrf.at[pl.ds(r0 * 128, 1024)], ab)
        pltpu.sync_copy(srcf32.at[pl.ds(r0 * 128, 1024)], sbf)
        pltpu.sync_copy(src2.at[pl.ds(r0, 8)], sb)
        pltpu.sync_copy(dst2.at[pl.ds(r0, 8)], db)

        pend = fire(0, gsets[0], gsems[0])
        for j in range(8):
            cur = pend
            if j < 7:
                pend = fire(j + 1, gsets[(j + 1) % 2], gsems[(j + 1) % 2])
            for cp in cur:
                cp.wait()
            gset = gsets[j % 2]

            @pl.loop(0, 128 // _L)
            def _(v):
                e = v * _L
                att = ab[pl.ds(j * 128 + e, _L)]
                srcv = plsc.bitcast(sbf[pl.ds(j * 128 + e, _L)], jnp.int32)
                g = jnp.zeros((_L,), jnp.int32)
                for ck in cums:
                    g = g + jnp.where(srcv >= ck, 1, 0)
                cb = plsc.load_gather(cbb, [g])
                lp = att * -0.5
                p = jnp.exp(lp)
                t = (gset[pl.ds(256 + e, _L)] + gset[pl.ds(640 + e, _L)]) * cb
                pn = p * t
                dx = gset[pl.ds(0 + e, _L)] - gset[pl.ds(384 + e, _L)]
                dy = gset[pl.ds(128 + e, _L)] - gset[pl.ds(512 + e, _L)]
                w1 = 1.0 + dx * dx + dy * dy
                qun = 1.0 / w1
                ekl = pn * (lp + _vlog(t * w1))
                fl = g * _L + lanes
                plsc.addupdate_scatter(accA, [fl], ekl)
                plsc.addupdate_scatter(accB, [fl], pn)
                plsc.addupdate_scatter(accQ, [fl], qun)

    pltpu.sync_copy(accA, outA.at[c, s])
    pltpu.sync_copy(accB, outB.at[c, s])
    pltpu.sync_copy(accQ, outQ.at[c, s])


def _fin_body(pA_ref, pB_ref, pQ_ref, o_ref):
    RA = pA_ref[...]
    RB = pB_ref[...]
    RQ = pQ_ref[...]
    col = lax.broadcasted_iota(jnp.int32, RA.shape, 1)
    gcol = col // _L
    ngraphs = 256 // _L
    total = jnp.float32(0.0)
    for g in range(ngraphs):
        m = gcol == g
        Ag = jnp.sum(jnp.where(m, RA, 0.0))
        Bg = jnp.sum(jnp.where(m, RB, 0.0))
        Qg = jnp.sum(jnp.where(m, RQ, 0.0))
        gk = jnp.where(Bg > 0.0, Ag + Bg * jnp.log(Qg), 0.0)
        total = total + gk
    o_ref[0, 0] = total / jnp.float32(ngraphs)


@jax.jit
def kernel(node_pos, full_edge_attr, full_edge_index, edge_index, batch, n):
    del edge_index
    n_nodes = node_pos.shape[0]
    n_edges = full_edge_index.shape[1]
    rows = n_edges // 128

    attrf = full_edge_attr[:, 0]
    src2 = full_edge_index[0].reshape(rows, 128)
    dst2 = full_edge_index[1].reshape(rows, 128)
    pad = _N_PAD - n_nodes
    posx = jnp.pad(node_pos[:, 0], (0, pad))
    posy = jnp.pad(node_pos[:, 1], (0, pad))
    cbv = 1.0 / (2.0 * n.astype(jnp.float32))
    cumv = jnp.concatenate([jnp.zeros((1,), jnp.int32),
                            jnp.cumsum(n.astype(jnp.int32))[:15]])

    mesh = plsc.VectorSubcoreMesh(
        core_axis_name="c", subcore_axis_name="s",
        num_cores=_NC, num_subcores=_NS)

    sc = pl.kernel(
        _sc_body,
        out_type=[jax.ShapeDtypeStruct((_NC, _NS, 256), jnp.float32)] * 3,
        mesh=mesh,
        compiler_params=pltpu.CompilerParams(needs_layout_passes=False),
        scratch_types=[
            pltpu.VMEM((_NPT,), jnp.float32),
            pltpu.VMEM((1024,), jnp.float32),
            pltpu.VMEM((8, 128), jnp.int32),
            pltpu.VMEM((8, 128), jnp.int32),
            pltpu.VMEM((1024,), jnp.float32),
            pltpu.VMEM((1024,), jnp.float32),
            pltpu.VMEM((_NPT,), jnp.float32),
            pltpu.VMEM((_NPT,), jnp.float32),
            pltpu.VMEM((768,), jnp.float32),
            pltpu.VMEM((768,), jnp.float32),
            pltpu.VMEM((256,), jnp.float32),
            pltpu.VMEM((256,), jnp.float32),
            pltpu.VMEM((256,), jnp.float32),
            pltpu.VMEM((16,), jnp.float32),
            pltpu.VMEM((16,), jnp.int32),
            pltpu.SemaphoreType.DMA,
            pltpu.SemaphoreType.DMA,
        ] + [pltpu.SemaphoreType.DMA] * 8
        + [pltpu.VMEM_SHARED((_N_PAD,), jnp.float32)] * 8
        + [
            pltpu.VMEM_SHARED((_N_PAD,), jnp.float32),
            pltpu.VMEM_SHARED((_N_PAD,), jnp.float32),
        ],
    )
    srcf32 = lax.bitcast_convert_type(full_edge_index[0], jnp.float32)
    pA, pB, pQ = sc(posx, posy, attrf, srcf32, src2, dst2, cbv, cumv)

    fin = pl.pallas_call(
        _fin_body,
        out_shape=jax.ShapeDtypeStruct((1, 1), jnp.float32),
        out_specs=pl.BlockSpec(memory_space=pltpu.SMEM),
    )
    res = fin(pA.reshape(_NC * _NS, 256),
              pB.reshape(_NC * _NS, 256),
              pQ.reshape(_NC * _NS, 256))
    return res[0, 0]


# speedup vs baseline: 162.5593x; 1.2749x over previous
import functools

import jax
import jax.numpy as jnp
from jax import lax
from jax.experimental import pallas as pl
from jax.experimental.pallas import tpu as pltpu
from jax.experimental.pallas import tpu_sc as plsc

_NC = 2
_NS = 16
_L = 16

_N_PAD = 102400
_NPT = _N_PAD // _NS

_LN2 = 0.6931471805599453
_SQRT2 = 1.4142135623730951


def _vlog(x):
    xi = plsc.bitcast(x, jnp.int32)
    e = (lax.shift_right_logical(xi, 23) & 0xFF) - 127
    mi = (xi & 0x007FFFFF) | 0x3F800000
    m = plsc.bitcast(mi, jnp.float32)
    big = m > _SQRT2
    m = jnp.where(big, m * 0.5, m)
    e = e + jnp.where(big, 1, 0)
    t = m - 1.0
    z = t / (2.0 + t)
    z2 = z * z
    lnm = 2.0 * z * (1.0 + z2 * (1.0 / 3.0 + z2 * (1.0 / 5.0 + z2 * (1.0 / 7.0))))
    return e.astype(jnp.float32) * _LN2 + lnm


def _split(total, parts, idx):
    base = total // parts
    rem = total % parts
    start = idx * base + jnp.minimum(idx, rem)
    cnt = base + jnp.where(idx < rem, 1, 0)
    return start, cnt


def _scatter_body(attrf, src2, dst2,
                  outP,
                  zb, ab, sb, db, pb, ssb, sdb,
                  sem_s0, sem_s1, sem_s2, sem_s3,
                  sem_d0, sem_d1, sem_d2, sem_d3,
                  ss0, ss1, ss2, ss3, sd0, sd1, sd2, sd3):
    sstabs = (ss0, ss1, ss2, ss3)
    sdtabs = (sd0, sd1, sd2, sd3)
    ssems = (sem_s0, sem_s1, sem_s2, sem_s3)
    dsems = (sem_d0, sem_d1, sem_d2, sem_d3)
    c = lax.axis_index("c")
    s = lax.axis_index("s")

    rows = src2.shape[0]
    nchunks = rows // 8

    z16f = jnp.zeros((_L,), jnp.float32)

    @pl.loop(0, _NPT // _L)
    def _(i):
        zb[pl.ds(i * _L, _L)] = z16f

    nb = s * _NPT
    nsl = pl.ds(nb, _NPT)
    for tab in sstabs + sdtabs:
        pltpu.sync_copy(zb, tab.at[nsl])
    plsc.subcore_barrier()

    cst, ccnt = _split(nchunks, _NC, c)
    tst, tcnt = _split(ccnt, _NS, s)
    st1 = cst + tst

    @pl.loop(st1, st1 + tcnt)
    def _(ch):
        r0 = ch * 8
        pltpu.sync_copy(attrf.at[pl.ds(r0 * 128, 1024)], ab)
        pltpu.sync_copy(src2.at[pl.ds(r0, 8)], sb)
        pltpu.sync_copy(dst2.at[pl.ds(r0, 8)], db)

        @pl.loop(0, 1024 // _L)
        def _(v):
            att = ab[pl.ds(v * _L, _L)]
            pb[pl.ds(v * _L, _L)] = jnp.exp(att * -0.5)

        sdesc = [None] * 4
        ddesc = [None] * 4
        for j in range(8):
            k = j % 4
            if sdesc[k] is not None:
                sdesc[k].wait()
                ddesc[k].wait()
            pj = pb.at[pl.ds(j * 128, 128)]
            sdesc[k] = pltpu.async_copy(pj, sstabs[k].at[sb.at[j]],
                                        ssems[k], add=True)
            ddesc[k] = pltpu.async_copy(pj, sdtabs[k].at[db.at[j]],
                                        dsems[k], add=True)
        for k in range(4):
            sdesc[k].wait()
            ddesc[k].wait()

    plsc.subcore_barrier()

    pltpu.sync_copy(ss0.at[nsl], ssb)
    pltpu.sync_copy(sd0.at[nsl], sdb)
    for tab, acc in ((ss1, ssb), (ss2, ssb), (ss3, ssb),
                     (sd1, sdb), (sd2, sdb), (sd3, sdb)):
        pltpu.sync_copy(tab.at[nsl], zb)

        @pl.loop(0, _NPT // _L)
        def _(v):
            sl = pl.ds(v * _L, _L)
            acc[sl] = acc[sl] + zb[sl]

    pltpu.sync_copy(ssb, outP.at[c, 0, nsl])
    pltpu.sync_copy(sdb, outP.at[c, 1, nsl])


def _main_body(posx, posy, attrf, srcf32, src2, dst2, cbv, cumv, partP,
               outA, outB, outQ,
               zb, ab, sb, db, sbf, ssb, sdb,
               g0, g1, accA, accB, accQ, cbb, cumb,
               semg0, semg1,
               ssT, sdT, pxT, pyT):
    c = lax.axis_index("c")
    s = lax.axis_index("s")
    w = c * _NS + s

    rows = src2.shape[0]
    nchunks = rows // 8

    lanes = lax.iota(jnp.int32, _L)
    z16f = jnp.zeros((_L,), jnp.float32)

    nb = s * _NPT
    nsl = pl.ds(nb, _NPT)
    pltpu.sync_copy(partP.at[0, 0, nsl], ssb)
    pltpu.sync_copy(partP.at[0, 1, nsl], sdb)
    for tab, acc in ((partP.at[1, 0, nsl], ssb), (partP.at[1, 1, nsl], sdb)):
        pltpu.sync_copy(tab, zb)

        @pl.loop(0, _NPT // _L)
        def _(v):
            sl = pl.ds(v * _L, _L)
            acc[sl] = acc[sl] + zb[sl]

    @pl.loop(0, _NPT // _L)
    def _(v):
        sl = pl.ds(v * _L, _L)
        ssb[sl] = 1.0 / ssb[sl]
        sdb[sl] = 1.0 / sdb[sl]

    pltpu.sync_copy(ssb, ssT.at[nsl])
    pltpu.sync_copy(sdb, sdT.at[nsl])
    pltpu.sync_copy(posx.at[nsl], pxT.at[nsl])
    pltpu.sync_copy(posy.at[nsl], pyT.at[nsl])

    @pl.loop(0, 256 // _L)
    def _(i):
        accA[pl.ds(i * _L, _L)] = z16f
        accB[pl.ds(i * _L, _L)] = z16f
        accQ[pl.ds(i * _L, _L)] = z16f

    pltpu.sync_copy(cbv, cbb)
    pltpu.sync_copy(cumv, cumb)
    plsc.subcore_barrier()

    cums = [plsc.load_gather(cumb, [jnp.full((_L,), k, jnp.int32)])
            for k in range(1, 16)]

    st2, cnt2 = _split(nchunks, _NC * _NS, w)

    gsets = (g0, g1)
    gsems = (semg0, semg1)

    def fire(j, gset, gsem):
        si = sb.at[j]
        di = db.at[j]
        return (
            pltpu.async_copy(pxT.at[si], gset.at[pl.ds(0, 128)], gsem),
            pltpu.async_copy(pyT.at[si], gset.at[pl.ds(128, 128)], gsem),
            pltpu.async_copy(ssT.at[si], gset.at[pl.ds(256, 128)], gsem),
            pltpu.async_copy(pxT.at[di], gset.at[pl.ds(384, 128)], gsem),
            pltpu.async_copy(pyT.at[di], gset.at[pl.ds(512, 128)], gsem),
            pltpu.async_copy(sdT.at[di], gset.at[pl.ds(640, 128)], gsem),
        )

    @pl.loop(st2, st2 + cnt2)
    def _(ch):
        r0 = ch * 8
        pltpu.sync_copy(attrf.at[pl.ds(r0 * 128, 1024)], ab)
        pltpu.sync_copy(srcf32.at[pl.ds(r0 * 128, 1024)], sbf)
        pltpu.sync_copy(src2.at[pl.ds(r0, 8)], sb)
        pltpu.sync_copy(dst2.at[pl.ds(r0, 8)], db)

        pend = fire(0, gsets[0], gsems[0])
        for j in range(8):
            cur = pend
            if j < 7:
                pend = fire(j + 1, gsets[(j + 1) % 2], gsems[(j + 1) % 2])
            for cp in cur:
                cp.wait()
            gset = gsets[j % 2]

            @pl.loop(0, 128 // _L)
            def _(v):
                e = v * _L
                att = ab[pl.ds(j * 128 + e, _L)]
                srcv = plsc.bitcast(sbf[pl.ds(j * 128 + e, _L)], jnp.int32)
                g = jnp.zeros((_L,), jnp.int32)
                for ck in cums:
                    g = g + jnp.where(srcv >= ck, 1, 0)
                cb = plsc.load_gather(cbb, [g])
                lp = att * -0.5
                p = jnp.exp(lp)
                t = (gset[pl.ds(256 + e, _L)] + gset[pl.ds(640 + e, _L)]) * cb
                pn = p * t
                dx = gset[pl.ds(0 + e, _L)] - gset[pl.ds(384 + e, _L)]
                dy = gset[pl.ds(128 + e, _L)] - gset[pl.ds(512 + e, _L)]
                w1 = 1.0 + dx * dx + dy * dy
                qun = 1.0 / w1
                ekl = pn * (lp + _vlog(t * w1))
                fl = g * _L + lanes
                plsc.addupdate_scatter(accA, [fl], ekl)
                plsc.addupdate_scatter(accB, [fl], pn)
                plsc.addupdate_scatter(accQ, [fl], qun)

    pltpu.sync_copy(accA, outA.at[c, s])
    pltpu.sync_copy(accB, outB.at[c, s])
    pltpu.sync_copy(accQ, outQ.at[c, s])


def _fin_body(pA_ref, pB_ref, pQ_ref, o_ref):
    RA = pA_ref[...]
    RB = pB_ref[...]
    RQ = pQ_ref[...]
    col = lax.broadcasted_iota(jnp.int32, RA.shape, 1)
    gcol = col // _L
    ngraphs = 256 // _L
    total = jnp.float32(0.0)
    for g in range(ngraphs):
        m = gcol == g
        Ag = jnp.sum(jnp.where(m, RA, 0.0))
        Bg = jnp.sum(jnp.where(m, RB, 0.0))
        Qg = jnp.sum(jnp.where(m, RQ, 0.0))
        gk = jnp.where(Bg > 0.0, Ag + Bg * jnp.log(Qg), 0.0)
        total = total + gk
    o_ref[0, 0] = total / jnp.float32(ngraphs)


@jax.jit
def kernel(node_pos, full_edge_attr, full_edge_index, edge_index, batch, n):
    del edge_index
    n_nodes = node_pos.shape[0]
    n_edges = full_edge_index.shape[1]
    rows = n_edges // 128

    attrf = full_edge_attr[:, 0]
    src2 = full_edge_index[0].reshape(rows, 128)
    dst2 = full_edge_index[1].reshape(rows, 128)
    pad = _N_PAD - n_nodes
    posx = jnp.pad(node_pos[:, 0], (0, pad))
    posy = jnp.pad(node_pos[:, 1], (0, pad))
    cbv = 1.0 / (2.0 * n.astype(jnp.float32))
    cumv = jnp.concatenate([jnp.zeros((1,), jnp.int32),
                            jnp.cumsum(n.astype(jnp.int32))[:15]])

    mesh = plsc.VectorSubcoreMesh(
        core_axis_name="c", subcore_axis_name="s",
        num_cores=_NC, num_subcores=_NS)

    scat = pl.kernel(
        _scatter_body,
        out_type=jax.ShapeDtypeStruct((_NC, 2, _N_PAD), jnp.float32),
        mesh=mesh,
        compiler_params=pltpu.CompilerParams(needs_layout_passes=False),
        scratch_types=[
            pltpu.VMEM((_NPT,), jnp.float32),
            pltpu.VMEM((1024,), jnp.float32),
            pltpu.VMEM((8, 128), jnp.int32),
            pltpu.VMEM((8, 128), jnp.int32),
            pltpu.VMEM((1024,), jnp.float32),
            pltpu.VMEM((_NPT,), jnp.float32),
            pltpu.VMEM((_NPT,), jnp.float32),
        ] + [pltpu.SemaphoreType.DMA] * 8
        + [pltpu.VMEM_SHARED((_N_PAD,), jnp.float32)] * 8,
    )
    partP = scat(attrf, src2, dst2)

    srcf32 = lax.bitcast_convert_type(full_edge_index[0], jnp.float32)
    main = pl.kernel(
        _main_body,
        out_type=[jax.ShapeDtypeStruct((_NC, _NS, 256), jnp.float32)] * 3,
        mesh=mesh,
        compiler_params=pltpu.CompilerParams(needs_layout_passes=False),
        scratch_types=[
            pltpu.VMEM((_NPT,), jnp.float32),
            pltpu.VMEM((1024,), jnp.float32),
            pltpu.VMEM((8, 128), jnp.int32),
            pltpu.VMEM((8, 128), jnp.int32),
            pltpu.VMEM((1024,), jnp.float32),
            pltpu.VMEM((_NPT,), jnp.float32),
            pltpu.VMEM((_NPT,), jnp.float32),
            pltpu.VMEM((768,), jnp.float32),
            pltpu.VMEM((768,), jnp.float32),
            pltpu.VMEM((256,), jnp.float32),
            pltpu.VMEM((256,), jnp.float32),
            pltpu.VMEM((256,), jnp.float32),
            pltpu.VMEM((16,), jnp.float32),
            pltpu.VMEM((16,), jnp.int32),
            pltpu.SemaphoreType.DMA,
            pltpu.SemaphoreType.DMA,
            pltpu.VMEM_SHARED((_N_PAD,), jnp.float32),
            pltpu.VMEM_SHARED((_N_PAD,), jnp.float32),
            pltpu.VMEM_SHARED((_N_PAD,), jnp.float32),
            pltpu.VMEM_SHARED((_N_PAD,), jnp.float32),
        ],
    )
    pA, pB, pQ = main(posx, posy, attrf, srcf32, src2, dst2, cbv, cumv, partP)

    fin = pl.pallas_call(
        _fin_body,
        out_shape=jax.ShapeDtypeStruct((1, 1), jnp.float32),
        out_specs=pl.BlockSpec(memory_space=pltpu.SMEM),
    )
    res = fin(pA.reshape(_NC * _NS, 256),
              pB.reshape(_NC * _NS, 256),
              pQ.reshape(_NC * _NS, 256))
    return res[0, 0]
